# baseline (speedup 1.0000x reference)
import jax
import jax.numpy as jnp
from jax import lax
from jax.experimental import pallas as pl
from jax.experimental.pallas import tpu as pltpu

_LOSS_LANES = 128
_VMEM_BUDGET = 56 * 1024 * 1024


def _round_up(x, m):
    return (x + m - 1) // m * m


def _lse_kernel(emb_ref, lse_ref):
    rows = emb_ref[...]
    m = jnp.max(rows, axis=-1, keepdims=True)
    lse_ref[...] = m + jnp.log(jnp.sum(jnp.exp(rows - m), axis=-1,
                                       keepdims=True))


def _gather_loss_kernel(tok_smem, tgt_smem, lse_smem, tok_ref, tgt_ref,
                        emb_ref, logits_ref, loss_ref):
    tm, v = logits_ref.shape
    ids = tok_ref[...]
    tgt = tgt_ref[...]
    col = lax.broadcasted_iota(jnp.int32, (tm, v), 1)

    onehot = (col == ids).astype(jnp.float32)
    logits = jnp.dot(onehot, emb_ref[...],
                     preferred_element_type=jnp.float32)
    logits_ref[...] = logits

    picked = jnp.sum(jnp.where(col == tgt, logits, 0.0),
                     axis=-1, keepdims=True)
    pick_sum = jnp.sum(picked)

    acc = [jnp.float32(0.0)] * 4
    for i in range(tm):
        t = tgt_smem[i]
        acc[i % 4] = acc[i % 4] + jnp.where(
            t >= 0, lse_smem[tok_smem[i]], 0.0)
    lse_sum = (acc[0] + acc[1]) + (acc[2] + acc[3])

    loss_ref[0] = jnp.full((1, _LOSS_LANES), lse_sum - pick_sum,
                           jnp.float32)


def kernel(x, emb, targets):
    B, T = x.shape
    V = emb.shape[0]
    assert emb.shape == (V, V)
    assert V % 128 == 0

    N = B * T
    row_tile = min(256, _round_up(N, 8))
    N_pad = _round_up(N, row_tile)
    num_tiles = N_pad // row_tile

    tok = jnp.pad(x.reshape(-1).astype(jnp.int32), (0, N_pad - N))
    tgt = jnp.pad(targets.reshape(-1).astype(jnp.int32),
                  (0, N_pad - N), constant_values=-1)

    vt = 256 if V % 256 == 0 else 128
    lse = pl.pallas_call(
        _lse_kernel,
        out_shape=jax.ShapeDtypeStruct((V, 1), jnp.float32),
        grid=(V // vt,),
        in_specs=[pl.BlockSpec((vt, V), lambda i: (i, 0))],
        out_specs=pl.BlockSpec((vt, 1), lambda i: (i, 0)),
        compiler_params=pltpu.CompilerParams(
            dimension_semantics=("parallel",),
            vmem_limit_bytes=_VMEM_BUDGET),
    )(emb)

    logits_pad, loss_tiles = pl.pallas_call(
        _gather_loss_kernel,
        out_shape=(
            jax.ShapeDtypeStruct((N_pad, V), jnp.float32),
            jax.ShapeDtypeStruct((num_tiles, 1, _LOSS_LANES), jnp.float32),
        ),
        grid=(num_tiles,),
        in_specs=[
            pl.BlockSpec((row_tile,), lambda i: (i,),
                         memory_space=pltpu.MemorySpace.SMEM),
            pl.BlockSpec((row_tile,), lambda i: (i,),
                         memory_space=pltpu.MemorySpace.SMEM),
            pl.BlockSpec(memory_space=pltpu.MemorySpace.SMEM),
            pl.BlockSpec((row_tile, 1), lambda i: (i, 0)),
            pl.BlockSpec((row_tile, 1), lambda i: (i, 0)),
            pl.BlockSpec(memory_space=pltpu.MemorySpace.VMEM),
        ],
        out_specs=(
            pl.BlockSpec((row_tile, V), lambda i: (i, 0)),
            pl.BlockSpec((1, 1, _LOSS_LANES), lambda i: (i, 0, 0)),
        ),
        compiler_params=pltpu.CompilerParams(
            dimension_semantics=("parallel",),
            vmem_limit_bytes=_VMEM_BUDGET),
    )(tok, tgt, lse.reshape(V), tok.reshape(N_pad, 1),
      tgt.reshape(N_pad, 1), emb)

    loss = jnp.sum(loss_tiles[:, 0, 0]) / N
    return logits_pad[:N], loss


# speedup vs baseline: 1.3987x; 1.3987x over previous
import jax
import jax.numpy as jnp
from jax import lax
from jax.experimental import pallas as pl
from jax.experimental.pallas import tpu as pltpu

_LOSS_LANES = 128
_VMEM_BUDGET = 56 * 1024 * 1024


def _round_up(x, m):
    return (x + m - 1) // m * m


def _lse_kernel(emb_ref, lse_ref):
    rows = emb_ref[...]
    m = jnp.max(rows, axis=-1, keepdims=True)
    lse_ref[...] = m + jnp.log(jnp.sum(jnp.exp(rows - m), axis=-1,
                                       keepdims=True))


def _gather_loss_kernel(tok_smem, tgt_smem, lse_smem, tgt_ref, emb_ref,
                        logits_ref, loss_ref):
    tm, v = logits_ref.shape

    def gather_group(g, carry):
        base = pl.multiple_of(g * 8, 8)
        rows = [emb_ref[tok_smem[base + j]] for j in range(8)]
        logits_ref[pl.ds(base, 8), :] = jnp.concatenate(rows, axis=0)
        return carry

    lax.fori_loop(0, tm // 8, gather_group, 0)

    logits = logits_ref[...]
    tgt = tgt_ref[...]
    col = lax.broadcasted_iota(jnp.int32, (tm, v), 1)
    pick_sum = jnp.sum(jnp.where(col == tgt, logits, 0.0))

    acc = [jnp.float32(0.0)] * 4
    for i in range(tm):
        t = tgt_smem[i]
        acc[i % 4] = acc[i % 4] + jnp.where(
            t >= 0, lse_smem[tok_smem[i]], 0.0)
    lse_sum = (acc[0] + acc[1]) + (acc[2] + acc[3])

    loss_ref[0] = jnp.full((1, _LOSS_LANES), lse_sum - pick_sum,
                           jnp.float32)


def kernel(x, emb, targets):
    B, T = x.shape
    V = emb.shape[0]
    assert emb.shape == (V, V)
    assert V % 128 == 0

    N = B * T
    row_tile = min(256, _round_up(N, 8))
    N_pad = _round_up(N, row_tile)
    num_tiles = N_pad // row_tile

    tok = jnp.pad(x.reshape(-1).astype(jnp.int32), (0, N_pad - N))
    tgt = jnp.pad(targets.reshape(-1).astype(jnp.int32),
                  (0, N_pad - N), constant_values=-1)
    emb3 = emb.reshape(V, 1, V)

    vt = 256 if V % 256 == 0 else 128
    lse = pl.pallas_call(
        _lse_kernel,
        out_shape=jax.ShapeDtypeStruct((V, 1), jnp.float32),
        grid=(V // vt,),
        in_specs=[pl.BlockSpec((vt, V), lambda i: (i, 0))],
        out_specs=pl.BlockSpec((vt, 1), lambda i: (i, 0)),
        compiler_params=pltpu.CompilerParams(
            dimension_semantics=("parallel",),
            vmem_limit_bytes=_VMEM_BUDGET),
    )(emb)

    logits_pad, loss_tiles = pl.pallas_call(
        _gather_loss_kernel,
        out_shape=(
            jax.ShapeDtypeStruct((N_pad, V), jnp.float32),
            jax.ShapeDtypeStruct((num_tiles, 1, _LOSS_LANES), jnp.float32),
        ),
        grid=(num_tiles,),
        in_specs=[
            pl.BlockSpec((row_tile,), lambda i: (i,),
                         memory_space=pltpu.MemorySpace.SMEM),
            pl.BlockSpec((row_tile,), lambda i: (i,),
                         memory_space=pltpu.MemorySpace.SMEM),
            pl.BlockSpec(memory_space=pltpu.MemorySpace.SMEM),
            pl.BlockSpec((row_tile, 1), lambda i: (i, 0)),
            pl.BlockSpec(memory_space=pltpu.MemorySpace.VMEM),
        ],
        out_specs=(
            pl.BlockSpec((row_tile, V), lambda i: (i, 0)),
            pl.BlockSpec((1, 1, _LOSS_LANES), lambda i: (i, 0, 0)),
        ),
        compiler_params=pltpu.CompilerParams(
            dimension_semantics=("parallel",),
            vmem_limit_bytes=_VMEM_BUDGET),
    )(tok, tgt, lse.reshape(V), tgt.reshape(N_pad, 1), emb3)

    loss = jnp.sum(loss_tiles[:, 0, 0]) / N
    return logits_pad[:N], loss
